# baseline (speedup 1.0000x reference)
import functools

import numpy as np

import jax
import jax.numpy as jnp
from jax import lax
from jax.experimental import pallas as pl
from jax.experimental.pallas import tpu as pltpu
from jax.experimental.pallas import tpu_sc as plsc

NLAYER = 3
C = 128
DE = 16
N = 10000
E = 320000
B = 128
VOCAB = 100

NC = 2
NS = 16
NW = NC * NS
EPT = E // NW
K = 40
NCHUNK = EPT // K
RB = 1000
NB = N // RB

_LOG1P = (9.975032552067553e-06, 0.9992354838332789, -0.4902307234234253,
          0.28527268109059317, -0.13158182508876734, 0.030449004538668306)


def _sc_edge_body(tdst, tsrc, efs, dsts, srcs, out,
                  idxd0, idxs0, idxd1, idxs1, isc0, isc1,
                  rd0, rs0, re0, rd1, rs1, re1, m0, m1, agg_sh,
                  isem0, isem1, gsem0, gsem1, wsem0, wsem1):
    c = lax.axis_index("c")
    s = lax.axis_index("s")
    w = c * NS + s
    tile_base = w * EPT

    def _zrow(r, carry):
        for q in range(C // 16):
            m0[r, pl.ds(q * 16, 16)] = jnp.zeros((16,), jnp.float32)
        return carry
    lax.fori_loop(0, K, _zrow, None)

    r0 = jnp.where(s == NS - 1, 9600, s * 640)
    nrow = jnp.where(s == NS - 1, 400, 640)

    def _zcp(t, carry):
        pltpu.sync_copy(m0, agg_sh.at[pl.ds(r0 + t * K, K)])
        return carry
    lax.fori_loop(0, nrow // K, _zcp, None)
    plsc.subcore_barrier()

    def _base(i):
        return jnp.minimum(tile_base + i * K, E - K)

    def _issue_idx(i, idxd, idxs, sem):
        b = _base(i)
        pltpu.async_copy(dsts.at[pl.ds(b, K)], idxd, sem)
        pltpu.async_copy(srcs.at[pl.ds(b, K)], idxs, sem)

    def _wait_idx(idxd, idxs, sem):
        pltpu.make_async_copy(dsts.at[pl.ds(0, K)], idxd, sem).wait()
        pltpu.make_async_copy(srcs.at[pl.ds(0, K)], idxs, sem).wait()

    def _issue_gather(i, idxd, idxs, rd, rs, re, sem):
        pltpu.async_copy(tdst.at[idxd], rd, sem)
        pltpu.async_copy(tsrc.at[idxs], rs, sem)
        pltpu.async_copy(efs.at[pl.ds(_base(i), K)], re, sem)

    def _wait_gather(idxd, idxs, rd, rs, re, sem):
        pltpu.make_async_copy(tdst.at[idxd], rd, sem).wait()
        pltpu.make_async_copy(tsrc.at[idxs], rs, sem).wait()
        pltpu.make_async_copy(efs.at[pl.ds(0, K)], re, sem).wait()

    def _compute(rd, rs, re, m_v):
        hmask = jnp.full((16,), -65536, jnp.int32)

        @plsc.parallel_loop(0, K, unroll=4)
        def _edge(j):
            for q in range(C // 16):
                sl = pl.ds(16 * q, 16)
                vd = rd[j, sl]
                vs = rs[j, sl]
                ve = re[j, sl]
                fh = (lax.bitcast_convert_type(vd << 16, jnp.float32)
                      + lax.bitcast_convert_type(vs << 16, jnp.float32)
                      + lax.bitcast_convert_type(ve << 16, jnp.float32))
                th = (lax.bitcast_convert_type(vd & hmask, jnp.float32)
                      + lax.bitcast_convert_type(vs & hmask, jnp.float32)
                      + lax.bitcast_convert_type(ve & hmask, jnp.float32))
                den = 1.0 + jnp.exp(-fh)
                u = jnp.exp(-jnp.abs(th))
                p = jnp.full((16,), _LOG1P[5], jnp.float32)
                for k in range(4, -1, -1):
                    p = p * u + _LOG1P[k]
                sp = jnp.maximum(th, 0.0) + p
                m_v[j, sl] = sp / den

    def _copy_idx(src_ref, dst_ref):
        for o in (0, 16, 24):
            dst_ref[pl.ds(o, 16)] = src_ref[pl.ds(o, 16)]

    _issue_idx(0, idxd0, idxs0, isem0)
    _wait_idx(idxd0, idxs0, isem0)
    _issue_gather(0, idxd0, idxs0, rd0, rs0, re0, gsem0)
    _issue_idx(1, idxd1, idxs1, isem1)

    def _pair(pi, carry):
        i0 = pi * 2
        _wait_idx(idxd1, idxs1, isem1)
        _issue_gather(i0 + 1, idxd1, idxs1, rd1, rs1, re1, gsem1)
        _wait_gather(idxd0, idxs0, rd0, rs0, re0, gsem0)

        @pl.when(pi > 0)
        def _():
            pltpu.make_async_copy(m0, agg_sh.at[isc0], wsem0).wait()
        _copy_idx(idxd0, isc0)
        _issue_idx(i0 + 2, idxd0, idxs0, isem0)
        _compute(rd0, rs0, re0, m0)
        pltpu.async_copy(m0, agg_sh.at[isc0], wsem0, add=True)
        _wait_idx(idxd0, idxs0, isem0)
        _issue_gather(i0 + 2, idxd0, idxs0, rd0, rs0, re0, gsem0)
        _wait_gather(idxd1, idxs1, rd1, rs1, re1, gsem1)

        @pl.when(pi > 0)
        def _():
            pltpu.make_async_copy(m1, agg_sh.at[isc1], wsem1).wait()
        _copy_idx(idxd1, isc1)
        _issue_idx(i0 + 3, idxd1, idxs1, isem1)
        _compute(rd1, rs1, re1, m1)
        pltpu.async_copy(m1, agg_sh.at[isc1], wsem1, add=True)
        return carry
    lax.fori_loop(0, NCHUNK // 2, _pair, None)

    _wait_gather(idxd0, idxs0, rd0, rs0, re0, gsem0)
    _wait_idx(idxd1, idxs1, isem1)
    pltpu.make_async_copy(m0, agg_sh.at[isc0], wsem0).wait()
    pltpu.make_async_copy(m1, agg_sh.at[isc1], wsem1).wait()
    plsc.subcore_barrier()

    def _ocp(t, carry):
        pltpu.sync_copy(agg_sh.at[pl.ds(r0 + t * K, K)],
                        out.at[pl.ds(c * N + r0 + t * K, K)])
        return carry
    lax.fori_loop(0, nrow // K, _ocp, None)


def _sc_edges(tdst, tsrc, efs_l, dsts, srcs):
    mesh = plsc.VectorSubcoreMesh(core_axis_name="c", subcore_axis_name="s")
    run = pl.kernel(
        _sc_edge_body,
        out_type=jax.ShapeDtypeStruct((NC * N, C), jnp.float32),
        mesh=mesh,
        scratch_types=[
            pltpu.VMEM((K,), jnp.int32),
            pltpu.VMEM((K,), jnp.int32),
            pltpu.VMEM((K,), jnp.int32),
            pltpu.VMEM((K,), jnp.int32),
            pltpu.VMEM((K,), jnp.int32),
            pltpu.VMEM((K,), jnp.int32),
            pltpu.VMEM((K, C), jnp.int32),
            pltpu.VMEM((K, C), jnp.int32),
            pltpu.VMEM((K, C), jnp.int32),
            pltpu.VMEM((K, C), jnp.int32),
            pltpu.VMEM((K, C), jnp.int32),
            pltpu.VMEM((K, C), jnp.int32),
            pltpu.VMEM((K, C), jnp.float32),
            pltpu.VMEM((K, C), jnp.float32),
            pltpu.VMEM_SHARED((N, C), jnp.float32),
            pltpu.SemaphoreType.DMA,
            pltpu.SemaphoreType.DMA,
            pltpu.SemaphoreType.DMA,
            pltpu.SemaphoreType.DMA,
            pltpu.SemaphoreType.DMA,
            pltpu.SemaphoreType.DMA,
        ],
    )
    return run(tdst, tsrc, efs_l, dsts, srcs)


def _pack_fs(f32f, f32s):
    lo = lax.bitcast_convert_type(f32f.astype(jnp.bfloat16),
                                  jnp.uint16).astype(jnp.uint32)
    hi = lax.bitcast_convert_type(f32s.astype(jnp.bfloat16),
                                  jnp.uint16).astype(jnp.uint32)
    return lax.bitcast_convert_type(lo | (hi << 16), jnp.int32)


def _embed_body(x_ref, emb_ref, wd_ref, ws_ref, h_ref, td_ref, ts_ref):
    iot = lax.broadcasted_iota(jnp.int32, (RB, C), 1)
    oh = (x_ref[...] == iot).astype(jnp.float32)
    h = jnp.dot(oh, emb_ref[...], preferred_element_type=jnp.float32,
                precision=lax.Precision.HIGHEST)
    h_ref[...] = h
    td = jnp.dot(h, wd_ref[...], preferred_element_type=jnp.float32,
                 precision=lax.Precision.HIGHEST)
    ts = jnp.dot(h, ws_ref[...], preferred_element_type=jnp.float32,
                 precision=lax.Precision.HIGHEST)
    td_ref[...] = _pack_fs(td[:, :C], td[:, C:])
    ts_ref[...] = _pack_fs(ts[:, :C], ts[:, C:])


def _embed(x2, embp, wd, ws):
    return pl.pallas_call(
        _embed_body,
        grid=(NB,),
        in_specs=[
            pl.BlockSpec((RB, 1), lambda i: (i, 0)),
            pl.BlockSpec((C, C), lambda i: (0, 0)),
            pl.BlockSpec((C, 2 * C), lambda i: (0, 0)),
            pl.BlockSpec((C, 2 * C), lambda i: (0, 0)),
        ],
        out_specs=[
            pl.BlockSpec((RB, C), lambda i: (i, 0)),
            pl.BlockSpec((RB, C), lambda i: (i, 0)),
            pl.BlockSpec((RB, C), lambda i: (i, 0)),
        ],
        out_shape=[
            jax.ShapeDtypeStruct((N, C), jnp.float32),
            jax.ShapeDtypeStruct((N, C), jnp.int32),
            jax.ShapeDtypeStruct((N, C), jnp.int32),
        ],
    )(x2, embp, wd, ws)


EB = 3200


def _efs_body(ea_ref, we_ref, bia_ref, out_ref):
    r = jnp.dot(ea_ref[...], we_ref[0], preferred_element_type=jnp.float32,
                precision=lax.Precision.HIGHEST) + bia_ref[0]
    out_ref[...] = _pack_fs(r[:, :C], r[:, C:])[None]


def _efs(edge_attr, we, bia):
    return pl.pallas_call(
        _efs_body,
        grid=(NLAYER, E // EB),
        in_specs=[
            pl.BlockSpec((EB, DE), lambda l, j: (j, 0)),
            pl.BlockSpec((1, DE, 2 * C), lambda l, j: (l, 0, 0)),
            pl.BlockSpec((1, 1, 2 * C), lambda l, j: (l, 0, 0)),
        ],
        out_specs=pl.BlockSpec((1, EB, C), lambda l, j: (l, j, 0)),
        out_shape=jax.ShapeDtypeStruct((NLAYER, E, C), jnp.int32),
    )(edge_attr, we, bia)


def _stats_body(p_ref, agg_ref, ssq_ref):
    a = p_ref[0] + p_ref[1]
    agg_ref[...] = a
    s1 = jnp.sum(a, axis=0, keepdims=True)
    s2 = jnp.sum(a * a, axis=0, keepdims=True)
    pad = jnp.zeros((6, C), jnp.float32)
    ssq_ref[...] = jnp.concatenate([s1, s2, pad], axis=0)[None]


def _stats(pagg):
    return pl.pallas_call(
        _stats_body,
        grid=(NB,),
        in_specs=[pl.BlockSpec((NC, RB, C), lambda i: (0, i, 0))],
        out_specs=[
            pl.BlockSpec((RB, C), lambda i: (i, 0)),
            pl.BlockSpec((1, 8, C), lambda i: (i, 0, 0)),
        ],
        out_shape=[
            jax.ShapeDtypeStruct((N, C), jnp.float32),
            jax.ShapeDtypeStruct((NB, 8, C), jnp.float32),
        ],
    )(pagg)


def _bn_res_relu(agg_ref, h_ref, ssq_ref, g_ref, b_ref):
    ssq = ssq_ref[...]
    mean = jnp.sum(ssq[:, 0:1, :], axis=0) / N
    var = jnp.sum(ssq[:, 1:2, :], axis=0) / N - mean * mean
    scale = g_ref[...] * lax.rsqrt(var + 1e-5)
    norm = (agg_ref[...] - mean) * scale + b_ref[...]
    return jnp.maximum(norm + h_ref[...], 0.0)


def _segmax_partial(hn, bat_ref):
    bb = bat_ref[...]
    parts = []
    for g in range(B):
        wv = jnp.where(bb == g, hn, -1e30)
        parts.append(jnp.max(wv, axis=0, keepdims=True))
    return jnp.concatenate(parts, axis=0)[None]


def _apply_body(agg_ref, h_ref, ssq_ref, g_ref, b_ref, bat_ref,
                wd_ref, ws_ref, hn_ref, td_ref, ts_ref, gfp_ref):
    hn = _bn_res_relu(agg_ref, h_ref, ssq_ref, g_ref, b_ref)
    hn_ref[...] = hn
    td = jnp.dot(hn, wd_ref[...], preferred_element_type=jnp.float32,
                 precision=lax.Precision.HIGHEST)
    ts = jnp.dot(hn, ws_ref[...], preferred_element_type=jnp.float32,
                 precision=lax.Precision.HIGHEST)
    td_ref[...] = _pack_fs(td[:, :C], td[:, C:])
    ts_ref[...] = _pack_fs(ts[:, :C], ts[:, C:])
    gfp_ref[...] = _segmax_partial(hn, bat_ref)


def _apply(agg, h, ssq, g, b, bat2, wd, ws):
    return pl.pallas_call(
        _apply_body,
        grid=(NB,),
        in_specs=[
            pl.BlockSpec((RB, C), lambda i: (i, 0)),
            pl.BlockSpec((RB, C), lambda i: (i, 0)),
            pl.BlockSpec((NB, 8, C), lambda i: (0, 0, 0)),
            pl.BlockSpec((1, C), lambda i: (0, 0)),
            pl.BlockSpec((1, C), lambda i: (0, 0)),
            pl.BlockSpec((RB, 1), lambda i: (i, 0)),
            pl.BlockSpec((C, 2 * C), lambda i: (0, 0)),
            pl.BlockSpec((C, 2 * C), lambda i: (0, 0)),
        ],
        out_specs=[
            pl.BlockSpec((RB, C), lambda i: (i, 0)),
            pl.BlockSpec((RB, C), lambda i: (i, 0)),
            pl.BlockSpec((RB, C), lambda i: (i, 0)),
            pl.BlockSpec((1, B, C), lambda i: (i, 0, 0)),
        ],
        out_shape=[
            jax.ShapeDtypeStruct((N, C), jnp.float32),
            jax.ShapeDtypeStruct((N, C), jnp.int32),
            jax.ShapeDtypeStruct((N, C), jnp.int32),
            jax.ShapeDtypeStruct((NB, B, C), jnp.float32),
        ],
    )(agg, h, ssq, g, b, bat2, wd, ws)


def _apply_last_body(agg_ref, h_ref, ssq_ref, g_ref, b_ref, bat_ref, gfp_ref):
    hn = _bn_res_relu(agg_ref, h_ref, ssq_ref, g_ref, b_ref)
    gfp_ref[...] = _segmax_partial(hn, bat_ref)


def _apply_last(agg, h, ssq, g, b, bat2):
    return pl.pallas_call(
        _apply_last_body,
        grid=(NB,),
        in_specs=[
            pl.BlockSpec((RB, C), lambda i: (i, 0)),
            pl.BlockSpec((RB, C), lambda i: (i, 0)),
            pl.BlockSpec((NB, 8, C), lambda i: (0, 0, 0)),
            pl.BlockSpec((1, C), lambda i: (0, 0)),
            pl.BlockSpec((1, C), lambda i: (0, 0)),
            pl.BlockSpec((RB, 1), lambda i: (i, 0)),
        ],
        out_specs=pl.BlockSpec((1, B, C), lambda i: (i, 0, 0)),
        out_shape=jax.ShapeDtypeStruct((NB, B, C), jnp.float32),
    )(agg, h, ssq, g, b, bat2)


def _head_body(g1_ref, g2_ref, g3_ref, w1_ref, b1_ref, w2_ref, b2_ref, o_ref):
    gf = (jnp.max(g1_ref[...], axis=0) + jnp.max(g2_ref[...], axis=0)
          + jnp.max(g3_ref[...], axis=0))
    x1 = jnp.dot(gf, w1_ref[...], preferred_element_type=jnp.float32,
                 precision=lax.Precision.HIGHEST)
    x1 = jnp.maximum(x1 + b1_ref[...], 0.0)
    o = jnp.dot(x1, w2_ref[...], preferred_element_type=jnp.float32,
                precision=lax.Precision.HIGHEST)
    o_ref[...] = o + b2_ref[...]


def _head(g1, g2, g3, w1, b1, w2p, b2):
    return pl.pallas_call(
        _head_body,
        out_shape=jax.ShapeDtypeStruct((B, C), jnp.float32),
    )(g1, g2, g3, w1, b1, w2p, b2)


def kernel(x, edge_index, edge_attr, batch, emb, Wf, bf, Ws, bs,
           gamma, beta, lin1_W, lin1_b, lin2_W, lin2_b):
    f32 = jnp.float32
    x2 = x.astype(jnp.int32).reshape(N, 1)
    src = edge_index[0].astype(jnp.int32)
    dst = edge_index[1].astype(jnp.int32)
    bat2 = batch.astype(jnp.int32).reshape(N, 1)
    embp = jnp.zeros((C, C), f32).at[:VOCAB].set(emb.astype(f32))

    wd = [jnp.concatenate([Wf[l, :C], Ws[l, :C]], axis=1)
          for l in range(NLAYER)]
    wsr = [jnp.concatenate([Wf[l, C:2 * C], Ws[l, C:2 * C]], axis=1)
           for l in range(NLAYER)]
    we = jnp.stack([jnp.concatenate([Wf[l, 2 * C:], Ws[l, 2 * C:]], axis=1)
                    for l in range(NLAYER)])
    bia = jnp.stack([jnp.concatenate([bf[l], bs[l]])[None]
                     for l in range(NLAYER)])

    h, td, ts = _embed(x2, embp, wd[0], wsr[0])
    efs = _efs(edge_attr.astype(f32), we, bia)

    gfps = []
    for l in range(NLAYER):
        pagg = _sc_edges(td, ts, efs[l], dst, src).reshape(NC, N, C)
        agg, ssq = _stats(pagg)
        g = gamma[l].reshape(1, C)
        b = beta[l].reshape(1, C)
        if l < NLAYER - 1:
            h, td, ts, gfp = _apply(agg, h, ssq, g, b, bat2,
                                    wd[l + 1], wsr[l + 1])
        else:
            gfp = _apply_last(agg, h, ssq, g, b, bat2)
        gfps.append(gfp)

    w2p = jnp.zeros((C, C), f32).at[:, :1].set(lin2_W.astype(f32))
    b2 = jnp.broadcast_to(lin2_b.astype(f32), (1, C))
    out = _head(gfps[0], gfps[1], gfps[2], lin1_W.astype(f32),
                lin1_b.reshape(1, C).astype(f32), w2p, b2)
    return out[:, 0]


# speedup vs baseline: 2.4697x; 2.4697x over previous
import functools

import numpy as np

import jax
import jax.numpy as jnp
from jax import lax
from jax.experimental import pallas as pl
from jax.experimental.pallas import tpu as pltpu
from jax.experimental.pallas import tpu_sc as plsc

NLAYER = 3
C = 128
DE = 16
N = 10000
E = 320000
B = 128
VOCAB = 100

NC = 2
NS = 16
NW = NC * NS
EPT = E // NW
K = 40
NCHUNK = EPT // K
RB = 1000
NB = N // RB

_LOG1P = (9.975032552067553e-06, 0.9992354838332789, -0.4902307234234253,
          0.28527268109059317, -0.13158182508876734, 0.030449004538668306)


def _sc_edge_body(tdst, tsrc, efs, dsts, srcs, out,
                  idxd0, idxs0, idxd1, idxs1, isc0, isc1,
                  rd0, rs0, re0, rd1, rs1, re1, m0, m1, agg_sh,
                  isem0, isem1, gsem0, gsem1, wsem0, wsem1):
    c = lax.axis_index("c")
    s = lax.axis_index("s")
    w = c * NS + s
    tile_base = w * EPT

    def _zrow(r, carry):
        for q in range(C // 16):
            m0[r, pl.ds(q * 16, 16)] = jnp.zeros((16,), jnp.float32)
        return carry
    lax.fori_loop(0, K, _zrow, None)

    r0 = jnp.where(s == NS - 1, 9600, s * 640)
    nrow = jnp.where(s == NS - 1, 400, 640)

    def _zcp(t, carry):
        pltpu.sync_copy(m0, agg_sh.at[pl.ds(r0 + t * K, K)])
        return carry
    lax.fori_loop(0, nrow // K, _zcp, None)
    plsc.subcore_barrier()

    def _base(i):
        return jnp.minimum(tile_base + i * K, E - K)

    def _issue_idx(i, idxd, idxs, sem):
        b = _base(i)
        pltpu.async_copy(dsts.at[pl.ds(b, K)], idxd, sem)
        pltpu.async_copy(srcs.at[pl.ds(b, K)], idxs, sem)

    def _wait_idx(idxd, idxs, sem):
        pltpu.make_async_copy(dsts.at[pl.ds(0, K)], idxd, sem).wait()
        pltpu.make_async_copy(srcs.at[pl.ds(0, K)], idxs, sem).wait()

    def _issue_gather(i, idxd, idxs, rd, rs, re, sem):
        pltpu.async_copy(tdst.at[idxd], rd, sem)
        pltpu.async_copy(tsrc.at[idxs], rs, sem)
        pltpu.async_copy(efs.at[pl.ds(_base(i), K)], re, sem)

    def _wait_gather(idxd, idxs, rd, rs, re, sem):
        pltpu.make_async_copy(tdst.at[idxd], rd, sem).wait()
        pltpu.make_async_copy(tsrc.at[idxs], rs, sem).wait()
        pltpu.make_async_copy(efs.at[pl.ds(0, K)], re, sem).wait()

    def _compute(rd, rs, re, m_v):
        hmask = jnp.full((16,), -65536, jnp.int32)

        @plsc.parallel_loop(0, K, unroll=1)
        def _edge(j):
            for q in range(C // 16):
                sl = pl.ds(16 * q, 16)
                vd = rd[j, sl]
                vs = rs[j, sl]
                ve = re[j, sl]
                fh = (lax.bitcast_convert_type(vd << 16, jnp.float32)
                      + lax.bitcast_convert_type(vs << 16, jnp.float32)
                      + lax.bitcast_convert_type(ve << 16, jnp.float32))
                th = (lax.bitcast_convert_type(vd & hmask, jnp.float32)
                      + lax.bitcast_convert_type(vs & hmask, jnp.float32)
                      + lax.bitcast_convert_type(ve & hmask, jnp.float32))
                den = 1.0 + jnp.exp(-fh)
                u = jnp.exp(-jnp.abs(th))
                p = jnp.full((16,), _LOG1P[5], jnp.float32)
                for k in range(4, -1, -1):
                    p = p * u + _LOG1P[k]
                sp = jnp.maximum(th, 0.0) + p
                m_v[j, sl] = sp / den

    def _copy_idx(src_ref, dst_ref):
        for o in (0, 16, 24):
            dst_ref[pl.ds(o, 16)] = src_ref[pl.ds(o, 16)]

    _issue_idx(0, idxd0, idxs0, isem0)
    _wait_idx(idxd0, idxs0, isem0)
    _issue_gather(0, idxd0, idxs0, rd0, rs0, re0, gsem0)
    _issue_idx(1, idxd1, idxs1, isem1)

    def _pair(pi, carry):
        i0 = pi * 2
        _wait_idx(idxd1, idxs1, isem1)
        _issue_gather(i0 + 1, idxd1, idxs1, rd1, rs1, re1, gsem1)
        _wait_gather(idxd0, idxs0, rd0, rs0, re0, gsem0)

        @pl.when(pi > 0)
        def _():
            pltpu.make_async_copy(m0, agg_sh.at[isc0], wsem0).wait()
        _copy_idx(idxd0, isc0)
        _issue_idx(i0 + 2, idxd0, idxs0, isem0)
        _compute(rd0, rs0, re0, m0)
        pltpu.async_copy(m0, agg_sh.at[isc0], wsem0, add=True)
        _wait_idx(idxd0, idxs0, isem0)
        _issue_gather(i0 + 2, idxd0, idxs0, rd0, rs0, re0, gsem0)
        _wait_gather(idxd1, idxs1, rd1, rs1, re1, gsem1)

        @pl.when(pi > 0)
        def _():
            pltpu.make_async_copy(m1, agg_sh.at[isc1], wsem1).wait()
        _copy_idx(idxd1, isc1)
        _issue_idx(i0 + 3, idxd1, idxs1, isem1)
        _compute(rd1, rs1, re1, m1)
        pltpu.async_copy(m1, agg_sh.at[isc1], wsem1, add=True)
        return carry
    lax.fori_loop(0, NCHUNK // 2, _pair, None)

    _wait_gather(idxd0, idxs0, rd0, rs0, re0, gsem0)
    _wait_idx(idxd1, idxs1, isem1)
    pltpu.make_async_copy(m0, agg_sh.at[isc0], wsem0).wait()
    pltpu.make_async_copy(m1, agg_sh.at[isc1], wsem1).wait()
    plsc.subcore_barrier()

    def _ocp(t, carry):
        pltpu.sync_copy(agg_sh.at[pl.ds(r0 + t * K, K)],
                        out.at[pl.ds(c * N + r0 + t * K, K)])
        return carry
    lax.fori_loop(0, nrow // K, _ocp, None)


def _sc_edges(tdst, tsrc, efs_l, dsts, srcs):
    mesh = plsc.VectorSubcoreMesh(core_axis_name="c", subcore_axis_name="s")
    run = pl.kernel(
        _sc_edge_body,
        out_type=jax.ShapeDtypeStruct((NC * N, C), jnp.float32),
        mesh=mesh,
        scratch_types=[
            pltpu.VMEM((K,), jnp.int32),
            pltpu.VMEM((K,), jnp.int32),
            pltpu.VMEM((K,), jnp.int32),
            pltpu.VMEM((K,), jnp.int32),
            pltpu.VMEM((K,), jnp.int32),
            pltpu.VMEM((K,), jnp.int32),
            pltpu.VMEM((K, C), jnp.int32),
            pltpu.VMEM((K, C), jnp.int32),
            pltpu.VMEM((K, C), jnp.int32),
            pltpu.VMEM((K, C), jnp.int32),
            pltpu.VMEM((K, C), jnp.int32),
            pltpu.VMEM((K, C), jnp.int32),
            pltpu.VMEM((K, C), jnp.float32),
            pltpu.VMEM((K, C), jnp.float32),
            pltpu.VMEM_SHARED((N, C), jnp.float32),
            pltpu.SemaphoreType.DMA,
            pltpu.SemaphoreType.DMA,
            pltpu.SemaphoreType.DMA,
            pltpu.SemaphoreType.DMA,
            pltpu.SemaphoreType.DMA,
            pltpu.SemaphoreType.DMA,
        ],
    )
    return run(tdst, tsrc, efs_l, dsts, srcs)


def _pack_fs(f32f, f32s):
    lo = lax.bitcast_convert_type(f32f.astype(jnp.bfloat16),
                                  jnp.uint16).astype(jnp.uint32)
    hi = lax.bitcast_convert_type(f32s.astype(jnp.bfloat16),
                                  jnp.uint16).astype(jnp.uint32)
    return lax.bitcast_convert_type(lo | (hi << 16), jnp.int32)


def _embed_body(x_ref, emb_ref, wd_ref, ws_ref, h_ref, td_ref, ts_ref):
    iot = lax.broadcasted_iota(jnp.int32, (RB, C), 1)
    oh = (x_ref[...] == iot).astype(jnp.float32)
    h = jnp.dot(oh, emb_ref[...], preferred_element_type=jnp.float32,
                precision=lax.Precision.HIGHEST)
    h_ref[...] = h
    td = jnp.dot(h, wd_ref[...], preferred_element_type=jnp.float32,
                 precision=lax.Precision.HIGHEST)
    ts = jnp.dot(h, ws_ref[...], preferred_element_type=jnp.float32,
                 precision=lax.Precision.HIGHEST)
    td_ref[...] = _pack_fs(td[:, :C], td[:, C:])
    ts_ref[...] = _pack_fs(ts[:, :C], ts[:, C:])


def _embed(x2, embp, wd, ws):
    return pl.pallas_call(
        _embed_body,
        grid=(NB,),
        in_specs=[
            pl.BlockSpec((RB, 1), lambda i: (i, 0)),
            pl.BlockSpec((C, C), lambda i: (0, 0)),
            pl.BlockSpec((C, 2 * C), lambda i: (0, 0)),
            pl.BlockSpec((C, 2 * C), lambda i: (0, 0)),
        ],
        out_specs=[
            pl.BlockSpec((RB, C), lambda i: (i, 0)),
            pl.BlockSpec((RB, C), lambda i: (i, 0)),
            pl.BlockSpec((RB, C), lambda i: (i, 0)),
        ],
        out_shape=[
            jax.ShapeDtypeStruct((N, C), jnp.float32),
            jax.ShapeDtypeStruct((N, C), jnp.int32),
            jax.ShapeDtypeStruct((N, C), jnp.int32),
        ],
    )(x2, embp, wd, ws)


EB = 3200


def _efs_body(ea_ref, we_ref, bia_ref, out_ref):
    r = jnp.dot(ea_ref[...], we_ref[0], preferred_element_type=jnp.float32,
                precision=lax.Precision.HIGHEST) + bia_ref[0]
    out_ref[...] = _pack_fs(r[:, :C], r[:, C:])[None]


def _efs(edge_attr, we, bia):
    return pl.pallas_call(
        _efs_body,
        grid=(NLAYER, E // EB),
        in_specs=[
            pl.BlockSpec((EB, DE), lambda l, j: (j, 0)),
            pl.BlockSpec((1, DE, 2 * C), lambda l, j: (l, 0, 0)),
            pl.BlockSpec((1, 1, 2 * C), lambda l, j: (l, 0, 0)),
        ],
        out_specs=pl.BlockSpec((1, EB, C), lambda l, j: (l, j, 0)),
        out_shape=jax.ShapeDtypeStruct((NLAYER, E, C), jnp.int32),
    )(edge_attr, we, bia)


def _stats_body(p_ref, agg_ref, ssq_ref):
    a = p_ref[0] + p_ref[1]
    agg_ref[...] = a
    s1 = jnp.sum(a, axis=0, keepdims=True)
    s2 = jnp.sum(a * a, axis=0, keepdims=True)
    pad = jnp.zeros((6, C), jnp.float32)
    ssq_ref[...] = jnp.concatenate([s1, s2, pad], axis=0)[None]


def _stats(pagg):
    return pl.pallas_call(
        _stats_body,
        grid=(NB,),
        in_specs=[pl.BlockSpec((NC, RB, C), lambda i: (0, i, 0))],
        out_specs=[
            pl.BlockSpec((RB, C), lambda i: (i, 0)),
            pl.BlockSpec((1, 8, C), lambda i: (i, 0, 0)),
        ],
        out_shape=[
            jax.ShapeDtypeStruct((N, C), jnp.float32),
            jax.ShapeDtypeStruct((NB, 8, C), jnp.float32),
        ],
    )(pagg)


def _bn_res_relu(agg_ref, h_ref, ssq_ref, g_ref, b_ref):
    ssq = ssq_ref[...]
    mean = jnp.sum(ssq[:, 0:1, :], axis=0) / N
    var = jnp.sum(ssq[:, 1:2, :], axis=0) / N - mean * mean
    scale = g_ref[...] * lax.rsqrt(var + 1e-5)
    norm = (agg_ref[...] - mean) * scale + b_ref[...]
    return jnp.maximum(norm + h_ref[...], 0.0)


def _segmax_partial(hn, bat_ref):
    bb = bat_ref[...]
    parts = []
    for g in range(B):
        wv = jnp.where(bb == g, hn, -1e30)
        parts.append(jnp.max(wv, axis=0, keepdims=True))
    return jnp.concatenate(parts, axis=0)[None]


def _apply_body(agg_ref, h_ref, ssq_ref, g_ref, b_ref, bat_ref,
                wd_ref, ws_ref, hn_ref, td_ref, ts_ref, gfp_ref):
    hn = _bn_res_relu(agg_ref, h_ref, ssq_ref, g_ref, b_ref)
    hn_ref[...] = hn
    td = jnp.dot(hn, wd_ref[...], preferred_element_type=jnp.float32,
                 precision=lax.Precision.HIGHEST)
    ts = jnp.dot(hn, ws_ref[...], preferred_element_type=jnp.float32,
                 precision=lax.Precision.HIGHEST)
    td_ref[...] = _pack_fs(td[:, :C], td[:, C:])
    ts_ref[...] = _pack_fs(ts[:, :C], ts[:, C:])
    gfp_ref[...] = _segmax_partial(hn, bat_ref)


def _apply(agg, h, ssq, g, b, bat2, wd, ws):
    return pl.pallas_call(
        _apply_body,
        grid=(NB,),
        in_specs=[
            pl.BlockSpec((RB, C), lambda i: (i, 0)),
            pl.BlockSpec((RB, C), lambda i: (i, 0)),
            pl.BlockSpec((NB, 8, C), lambda i: (0, 0, 0)),
            pl.BlockSpec((1, C), lambda i: (0, 0)),
            pl.BlockSpec((1, C), lambda i: (0, 0)),
            pl.BlockSpec((RB, 1), lambda i: (i, 0)),
            pl.BlockSpec((C, 2 * C), lambda i: (0, 0)),
            pl.BlockSpec((C, 2 * C), lambda i: (0, 0)),
        ],
        out_specs=[
            pl.BlockSpec((RB, C), lambda i: (i, 0)),
            pl.BlockSpec((RB, C), lambda i: (i, 0)),
            pl.BlockSpec((RB, C), lambda i: (i, 0)),
            pl.BlockSpec((1, B, C), lambda i: (i, 0, 0)),
        ],
        out_shape=[
            jax.ShapeDtypeStruct((N, C), jnp.float32),
            jax.ShapeDtypeStruct((N, C), jnp.int32),
            jax.ShapeDtypeStruct((N, C), jnp.int32),
            jax.ShapeDtypeStruct((NB, B, C), jnp.float32),
        ],
    )(agg, h, ssq, g, b, bat2, wd, ws)


def _apply_last_body(agg_ref, h_ref, ssq_ref, g_ref, b_ref, bat_ref, gfp_ref):
    hn = _bn_res_relu(agg_ref, h_ref, ssq_ref, g_ref, b_ref)
    gfp_ref[...] = _segmax_partial(hn, bat_ref)


def _apply_last(agg, h, ssq, g, b, bat2):
    return pl.pallas_call(
        _apply_last_body,
        grid=(NB,),
        in_specs=[
            pl.BlockSpec((RB, C), lambda i: (i, 0)),
            pl.BlockSpec((RB, C), lambda i: (i, 0)),
            pl.BlockSpec((NB, 8, C), lambda i: (0, 0, 0)),
            pl.BlockSpec((1, C), lambda i: (0, 0)),
            pl.BlockSpec((1, C), lambda i: (0, 0)),
            pl.BlockSpec((RB, 1), lambda i: (i, 0)),
        ],
        out_specs=pl.BlockSpec((1, B, C), lambda i: (i, 0, 0)),
        out_shape=jax.ShapeDtypeStruct((NB, B, C), jnp.float32),
    )(agg, h, ssq, g, b, bat2)


def _head_body(g1_ref, g2_ref, g3_ref, w1_ref, b1_ref, w2_ref, b2_ref, o_ref):
    gf = (jnp.max(g1_ref[...], axis=0) + jnp.max(g2_ref[...], axis=0)
          + jnp.max(g3_ref[...], axis=0))
    x1 = jnp.dot(gf, w1_ref[...], preferred_element_type=jnp.float32,
                 precision=lax.Precision.HIGHEST)
    x1 = jnp.maximum(x1 + b1_ref[...], 0.0)
    o = jnp.dot(x1, w2_ref[...], preferred_element_type=jnp.float32,
                precision=lax.Precision.HIGHEST)
    o_ref[...] = o + b2_ref[...]


def _head(g1, g2, g3, w1, b1, w2p, b2):
    return pl.pallas_call(
        _head_body,
        out_shape=jax.ShapeDtypeStruct((B, C), jnp.float32),
    )(g1, g2, g3, w1, b1, w2p, b2)


def kernel(x, edge_index, edge_attr, batch, emb, Wf, bf, Ws, bs,
           gamma, beta, lin1_W, lin1_b, lin2_W, lin2_b):
    f32 = jnp.float32
    x2 = x.astype(jnp.int32).reshape(N, 1)
    src = edge_index[0].astype(jnp.int32)
    dst = edge_index[1].astype(jnp.int32)
    bat2 = batch.astype(jnp.int32).reshape(N, 1)
    embp = jnp.zeros((C, C), f32).at[:VOCAB].set(emb.astype(f32))

    wd = [jnp.concatenate([Wf[l, :C], Ws[l, :C]], axis=1)
          for l in range(NLAYER)]
    wsr = [jnp.concatenate([Wf[l, C:2 * C], Ws[l, C:2 * C]], axis=1)
           for l in range(NLAYER)]
    we = jnp.stack([jnp.concatenate([Wf[l, 2 * C:], Ws[l, 2 * C:]], axis=1)
                    for l in range(NLAYER)])
    bia = jnp.stack([jnp.concatenate([bf[l], bs[l]])[None]
                     for l in range(NLAYER)])

    h, td, ts = _embed(x2, embp, wd[0], wsr[0])
    efs = _efs(edge_attr.astype(f32), we, bia)

    gfps = []
    for l in range(NLAYER):
        pagg = _sc_edges(td, ts, efs[l], dst, src).reshape(NC, N, C)
        agg, ssq = _stats(pagg)
        g = gamma[l].reshape(1, C)
        b = beta[l].reshape(1, C)
        if l < NLAYER - 1:
            h, td, ts, gfp = _apply(agg, h, ssq, g, b, bat2,
                                    wd[l + 1], wsr[l + 1])
        else:
            gfp = _apply_last(agg, h, ssq, g, b, bat2)
        gfps.append(gfp)

    w2p = jnp.zeros((C, C), f32).at[:, :1].set(lin2_W.astype(f32))
    b2 = jnp.broadcast_to(lin2_b.astype(f32), (1, C))
    out = _head(gfps[0], gfps[1], gfps[2], lin1_W.astype(f32),
                lin1_b.reshape(1, C).astype(f32), w2p, b2)
    return out[:, 0]
